# baseline (speedup 1.0000x reference)
import functools

import jax
import jax.numpy as jnp
from jax import lax
from jax.experimental import pallas as pl
from jax.experimental.pallas import tpu as pltpu
from jax.experimental.pallas import tpu_sc as plsc

N_NODES = 10000
N_EDGES = 320000
DIM = 128

NUM_CORES = 2
NUM_SUBCORES = 16
NW = NUM_CORES * NUM_SUBCORES

CHUNK = 128
NCH = 80
IB = 8
NBLK = NCH // IB
PAD_E = NW * NCH * CHUNK
TRASH = 1024
NACC = N_NODES + TRASH

INIT_PT = NACC // NUM_SUBCORES // 8 * 8
INIT_REM_OFF = INIT_PT * NUM_SUBCORES
OUT_PT = N_NODES // NUM_SUBCORES // 8 * 8
OUT_REM_OFF = OUT_PT * NUM_SUBCORES


def _sc_segment_sum(x, src_p, dst_p, zeros2d, zeros1d):
  mesh = plsc.VectorSubcoreMesh(
      core_axis_name="c", subcore_axis_name="s",
      num_cores=NUM_CORES, num_subcores=NUM_SUBCORES)

  @functools.partial(
      pl.kernel,
      out_type=[
          jax.ShapeDtypeStruct((NUM_CORES, N_NODES, DIM), jnp.float32),
          jax.ShapeDtypeStruct((NUM_CORES, NACC), jnp.float32),
      ],
      mesh=mesh,
      scratch_types=[
          pltpu.VMEM((NCH, CHUNK), jnp.int32),
          pltpu.VMEM((NCH, CHUNK), jnp.int32),
          pltpu.VMEM((CHUNK, DIM), jnp.float32),
          pltpu.VMEM((CHUNK,), jnp.float32),
          pltpu.VMEM_SHARED((NACC, DIM), jnp.float32),
          pltpu.VMEM_SHARED((NACC,), jnp.float32),
          pltpu.SemaphoreType.DMA,
      ],
  )
  def k(x_hbm, src_hbm, dst_hbm, z2_hbm, z1_hbm, parts_hbm, degs_hbm,
        idx_d, idx_s, rows, ones_v, acc_sh, deg_sh, gsem):
    c = lax.axis_index("c")
    s = lax.axis_index("s")
    w = s * NUM_CORES + c

    for j in range(CHUNK // 16):
      ones_v[pl.ds(j * 16, 16)] = jnp.ones((16,), jnp.float32)

    pltpu.sync_copy(src_hbm.at[w], idx_s)
    pltpu.sync_copy(dst_hbm.at[w], idx_d)
    pltpu.sync_copy(z2_hbm.at[pl.ds(s * INIT_PT, INIT_PT)],
                    acc_sh.at[pl.ds(s * INIT_PT, INIT_PT)])

    @pl.when(s == 0)
    def _():
      rem = NACC - INIT_REM_OFF
      pltpu.sync_copy(z2_hbm.at[pl.ds(INIT_REM_OFF, rem)],
                      acc_sh.at[pl.ds(INIT_REM_OFF, rem)])
      pltpu.sync_copy(z1_hbm, deg_sh)

    plsc.subcore_barrier()

    def body(j, carry):
      pltpu.async_copy(x_hbm.at[idx_d.at[j]], rows, gsem).wait()
      pltpu.sync_copy(rows, acc_sh.at[idx_s.at[j]], add=True)
      pltpu.sync_copy(ones_v, deg_sh.at[idx_s.at[j]], add=True)
      return carry

    lax.fori_loop(0, NCH, body, 0)

    plsc.subcore_barrier()

    row0 = s * OUT_PT
    pltpu.sync_copy(acc_sh.at[pl.ds(row0, OUT_PT)],
                    parts_hbm.at[c, pl.ds(row0, OUT_PT)])

    @pl.when(s == 0)
    def _():
      rem = N_NODES - OUT_REM_OFF
      pltpu.sync_copy(acc_sh.at[pl.ds(OUT_REM_OFF, rem)],
                      parts_hbm.at[c, pl.ds(OUT_REM_OFF, rem)])
      pltpu.sync_copy(deg_sh, degs_hbm.at[c])

  return k(x, src_p, dst_p, zeros2d, zeros1d)


BLK = 2000


def _tc_combine(x, parts, degs_t, wst, bs, wnt, bn):
  def body(x_ref, p_ref, d_ref, ws_ref, bs_ref, wn_ref, bn_ref, o_ref):
    xb = x_ref[...]
    sm = jnp.dot(xb, ws_ref[...], preferred_element_type=jnp.float32)
    sm = sm + bs_ref[...]
    psum = p_ref[0] + p_ref[1]
    deg = d_ref[:, 0:1] + d_ref[:, 1:2]
    mean = psum / jnp.maximum(deg, 1.0)
    nm = jnp.dot(mean, wn_ref[...], preferred_element_type=jnp.float32)
    nm = jnp.where(deg > 0.0, nm + bn_ref[...], 0.0)
    o_ref[...] = jnp.maximum(sm + nm, 0.0)

  return pl.pallas_call(
      body,
      grid=(N_NODES // BLK,),
      in_specs=[
          pl.BlockSpec((BLK, DIM), lambda i: (i, 0)),
          pl.BlockSpec((NUM_CORES, BLK, DIM), lambda i: (0, i, 0)),
          pl.BlockSpec((BLK, NUM_CORES), lambda i: (i, 0)),
          pl.BlockSpec((DIM, DIM), lambda i: (0, 0)),
          pl.BlockSpec((1, DIM), lambda i: (0, 0)),
          pl.BlockSpec((DIM, DIM), lambda i: (0, 0)),
          pl.BlockSpec((1, DIM), lambda i: (0, 0)),
      ],
      out_specs=pl.BlockSpec((BLK, DIM), lambda i: (i, 0)),
      out_shape=jax.ShapeDtypeStruct((N_NODES, DIM), jnp.float32),
  )(x, parts, degs_t, wst, bs, wnt, bn)


def kernel(x, edge_index, W_self, b_self, W_neigh, b_neigh):
  src = edge_index[0]
  dst = edge_index[1]
  pad = PAD_E - N_EDGES
  trash = N_NODES + (jnp.arange(pad, dtype=jnp.int32) % TRASH)
  src_p = (jnp.concatenate([src, trash])
           .reshape(NCH, NW, CHUNK).transpose(1, 0, 2))
  dst_p = (jnp.concatenate([dst, jnp.zeros((pad,), jnp.int32)])
           .reshape(NCH, NW, CHUNK).transpose(1, 0, 2))
  zeros2d = jnp.zeros((NACC, DIM), jnp.float32)
  zeros1d = jnp.zeros((NACC,), jnp.float32)
  parts, degs = _sc_segment_sum(x, src_p, dst_p, zeros2d, zeros1d)
  degs = degs[:, :N_NODES]
  return _tc_combine(x, parts, degs.T, W_self.T, b_self[None, :],
                     W_neigh.T, b_neigh[None, :])


# speedup vs baseline: 1.0543x; 1.0543x over previous
import functools

import jax
import jax.numpy as jnp
from jax import lax
from jax.experimental import pallas as pl
from jax.experimental.pallas import tpu as pltpu
from jax.experimental.pallas import tpu_sc as plsc

N_NODES = 10000
N_EDGES = 320000
DIM = 128

NUM_CORES = 2
NUM_SUBCORES = 16
NW = NUM_CORES * NUM_SUBCORES

CHUNK = 128
NCH = 80
IB = 8
NBLK = NCH // IB
PAD_E = NW * NCH * CHUNK
TRASH = 8
NACC = N_NODES + TRASH

INIT_PT = NACC // NUM_SUBCORES // 8 * 8
INIT_REM_OFF = INIT_PT * NUM_SUBCORES
OUT_PT = N_NODES // NUM_SUBCORES // 8 * 8
OUT_REM_OFF = OUT_PT * NUM_SUBCORES


def _sc_segment_sum(x, src_p, dst_p, zeros2d, zeros1d):
  mesh = plsc.VectorSubcoreMesh(
      core_axis_name="c", subcore_axis_name="s",
      num_cores=NUM_CORES, num_subcores=NUM_SUBCORES)

  @functools.partial(
      pl.kernel,
      out_type=[
          jax.ShapeDtypeStruct((NUM_CORES, N_NODES, DIM), jnp.float32),
          jax.ShapeDtypeStruct((NUM_CORES, NACC), jnp.float32),
      ],
      mesh=mesh,
      scratch_types=[
          pltpu.VMEM((CHUNK,), jnp.int32),
          pltpu.VMEM((CHUNK,), jnp.int32),
          pltpu.VMEM((CHUNK,), jnp.int32),
          pltpu.VMEM((CHUNK,), jnp.int32),
          pltpu.VMEM((CHUNK, DIM), jnp.float32),
          pltpu.VMEM((CHUNK, DIM), jnp.float32),
          pltpu.VMEM((CHUNK,), jnp.float32),
          pltpu.VMEM_SHARED((NACC, DIM), jnp.float32),
          pltpu.VMEM_SHARED((NACC,), jnp.float32),
          pltpu.SemaphoreType.DMA((2,)),
      ],
  )
  def k(x_hbm, src_hbm, dst_hbm, z2_hbm, z1_hbm, parts_hbm, degs_hbm,
        idx_d0, idx_d1, idx_s0, idx_s1, rows0, rows1, ones_v,
        acc_sh, deg_sh, gsem):
    c = lax.axis_index("c")
    s = lax.axis_index("s")
    w = s * NUM_CORES + c
    idx_d = [idx_d0, idx_d1]
    idx_s = [idx_s0, idx_s1]
    rows = [rows0, rows1]

    for j in range(CHUNK // 16):
      ones_v[pl.ds(j * 16, 16)] = jnp.ones((16,), jnp.float32)

    pltpu.sync_copy(z2_hbm.at[pl.ds(s * INIT_PT, INIT_PT)],
                    acc_sh.at[pl.ds(s * INIT_PT, INIT_PT)])

    @pl.when(s == 0)
    def _():
      rem = NACC - INIT_REM_OFF
      pltpu.sync_copy(z2_hbm.at[pl.ds(INIT_REM_OFF, rem)],
                      acc_sh.at[pl.ds(INIT_REM_OFF, rem)])
      pltpu.sync_copy(z1_hbm, deg_sh)

    def load_idx(j, b):
      pltpu.sync_copy(dst_hbm.at[w, j], idx_d[b])
      pltpu.sync_copy(src_hbm.at[w, j], idx_s[b])

    def gather(j, b):
      pltpu.async_copy(x_hbm.at[idx_d[b]], rows[b], gsem.at[b])

    def wait_gather(b):
      pltpu.make_async_copy(x_hbm.at[pl.ds(0, CHUNK)], rows[b],
                            gsem.at[b]).wait()

    def scatter(b):
      pltpu.sync_copy(rows[b], acc_sh.at[idx_s[b]], add=True)
      pltpu.sync_copy(ones_v, deg_sh.at[idx_s[b]], add=True)

    plsc.subcore_barrier()

    load_idx(0, 0)
    gather(0, 0)

    def body(k2, carry):
      for half in range(2):
        b = half
        j = 2 * k2 + half

        @pl.when(j + 1 < NCH)
        def _():
          load_idx(j + 1, 1 - b)
          gather(j + 1, 1 - b)

        wait_gather(b)
        scatter(b)

      return carry

    lax.fori_loop(0, NCH // 2, body, 0)

    plsc.subcore_barrier()

    row0 = s * OUT_PT
    pltpu.sync_copy(acc_sh.at[pl.ds(row0, OUT_PT)],
                    parts_hbm.at[c, pl.ds(row0, OUT_PT)])

    @pl.when(s == 0)
    def _():
      rem = N_NODES - OUT_REM_OFF
      pltpu.sync_copy(acc_sh.at[pl.ds(OUT_REM_OFF, rem)],
                      parts_hbm.at[c, pl.ds(OUT_REM_OFF, rem)])
      pltpu.sync_copy(deg_sh, degs_hbm.at[c])

  return k(x, src_p, dst_p, zeros2d, zeros1d)


BLK = 2000


def _tc_combine(x, parts, degs_t, wst, bs, wnt, bn):
  def body(x_ref, p_ref, d_ref, ws_ref, bs_ref, wn_ref, bn_ref, o_ref):
    xb = x_ref[...]
    sm = jnp.dot(xb, ws_ref[...], preferred_element_type=jnp.float32)
    sm = sm + bs_ref[...]
    psum = p_ref[0] + p_ref[1]
    deg = d_ref[:, 0:1] + d_ref[:, 1:2]
    mean = psum / jnp.maximum(deg, 1.0)
    nm = jnp.dot(mean, wn_ref[...], preferred_element_type=jnp.float32)
    nm = jnp.where(deg > 0.0, nm + bn_ref[...], 0.0)
    o_ref[...] = jnp.maximum(sm + nm, 0.0)

  return pl.pallas_call(
      body,
      grid=(N_NODES // BLK,),
      in_specs=[
          pl.BlockSpec((BLK, DIM), lambda i: (i, 0)),
          pl.BlockSpec((NUM_CORES, BLK, DIM), lambda i: (0, i, 0)),
          pl.BlockSpec((BLK, NUM_CORES), lambda i: (i, 0)),
          pl.BlockSpec((DIM, DIM), lambda i: (0, 0)),
          pl.BlockSpec((1, DIM), lambda i: (0, 0)),
          pl.BlockSpec((DIM, DIM), lambda i: (0, 0)),
          pl.BlockSpec((1, DIM), lambda i: (0, 0)),
      ],
      out_specs=pl.BlockSpec((BLK, DIM), lambda i: (i, 0)),
      out_shape=jax.ShapeDtypeStruct((N_NODES, DIM), jnp.float32),
  )(x, parts, degs_t, wst, bs, wnt, bn)


def kernel(x, edge_index, W_self, b_self, W_neigh, b_neigh):
  src = edge_index[0]
  dst = edge_index[1]
  pad = PAD_E - N_EDGES
  trash = N_NODES + (jnp.arange(pad, dtype=jnp.int32) % TRASH)
  src_p = (jnp.concatenate([src, trash])
           .reshape(NCH, NW, CHUNK).transpose(1, 0, 2))
  dst_p = (jnp.concatenate([dst, jnp.zeros((pad,), jnp.int32)])
           .reshape(NCH, NW, CHUNK).transpose(1, 0, 2))
  zeros2d = jnp.zeros((NACC, DIM), jnp.float32)
  zeros1d = jnp.zeros((NACC,), jnp.float32)
  parts, degs = _sc_segment_sum(x, src_p, dst_p, zeros2d, zeros1d)
  degs = degs[:, :N_NODES]
  return _tc_combine(x, parts, degs.T, W_self.T, b_self[None, :],
                     W_neigh.T, b_neigh[None, :])
